# baseline (speedup 1.0000x reference)
import functools

import jax
import jax.numpy as jnp
from jax import lax
from jax.experimental import pallas as pl
from jax.experimental.pallas import tpu as pltpu
from jax.experimental.pallas import tpu_sc as plsc

NC = 2
NS = 16
CH = 128
LANES = 16


def _cdiv(a, b):
    return (a + b - 1) // b


@functools.cache
def _deg_kernel(E, E_pad, P, rpt):
    mesh = plsc.VectorSubcoreMesh(core_axis_name="c", subcore_axis_name="s")
    nreal = E // CH
    nmax = _cdiv(nreal, NS * 8) * 8
    assert nmax * NS <= E_pad // CH

    @functools.partial(
        pl.kernel,
        out_type=(jax.ShapeDtypeStruct((P,), jnp.float32),
                  jax.ShapeDtypeStruct((P,), jnp.float32)),
        mesh=mesh,
        scratch_types=[
            pltpu.VMEM((nmax, CH), jnp.int32),
            pltpu.VMEM((CH,), jnp.float32),
            pltpu.VMEM((rpt,), jnp.float32),
            pltpu.VMEM_SHARED((P,), jnp.float32),
        ],
    )
    def k(src_h, dst_h, dout_h, din_h, idx_v, ones_v, stage_v, acc_s):
        c = lax.axis_index("c")
        s = lax.axis_index("s")
        for j in range(CH // LANES):
            ones_v[pl.ds(j * LANES, LANES)] = jnp.ones((LANES,), jnp.float32)

        def zero_body(i, carry):
            stage_v[pl.ds(i * LANES, LANES)] = jnp.zeros((LANES,), jnp.float32)
            return carry

        lax.fori_loop(0, rpt // LANES, zero_body, 0)
        pltpu.sync_copy(stage_v, acc_s.at[pl.ds(s * rpt, rpt)])
        plsc.subcore_barrier()
        start = s * nmax
        cnt = jnp.clip(nreal - s * nmax, 0, nmax)

        @pl.when(c == 0)
        def _():
            pltpu.sync_copy(src_h.at[pl.ds(start, nmax)], idx_v)

            def body(i, carry):
                pltpu.sync_copy(ones_v, acc_s.at[idx_v.at[i]], add=True)
                return carry

            lax.fori_loop(0, cnt, body, 0)

        @pl.when(c == 1)
        def _():
            pltpu.sync_copy(dst_h.at[pl.ds(start, nmax)], idx_v)

            def body(i, carry):
                pltpu.sync_copy(ones_v, acc_s.at[idx_v.at[i]], add=True)
                return carry

            lax.fori_loop(0, cnt, body, 0)

        plsc.subcore_barrier()
        pltpu.sync_copy(acc_s.at[pl.ds(s * rpt, rpt)], stage_v)

        @pl.when(c == 0)
        def _():
            pltpu.sync_copy(stage_v, dout_h.at[pl.ds(s * rpt, rpt)])

        @pl.when(c == 1)
        def _():
            pltpu.sync_copy(stage_v, din_h.at[pl.ds(s * rpt, rpt)])

    return k


@functools.cache
def _prop_kernel(E_pad, N, P, D, rpt):
    mesh = plsc.VectorSubcoreMesh(core_axis_name="c", subcore_axis_name="s")
    per_worker = E_pad // (NC * NS)
    nch = per_worker // CH
    assert nch % 2 == 0 and nch >= 4

    @functools.partial(
        pl.kernel,
        out_type=(jax.ShapeDtypeStruct((P, D), jnp.float32),
                  jax.ShapeDtypeStruct((P, D), jnp.float32)),
        mesh=mesh,
        scratch_types=[
            pltpu.VMEM((nch // 2, CH), jnp.int32),
            pltpu.VMEM((nch // 2, CH), jnp.int32),
            pltpu.VMEM((CH, D), jnp.float32),
            pltpu.VMEM((CH, D), jnp.float32),
            pltpu.VMEM_SHARED((P, D), jnp.float32),
            pltpu.SemaphoreType.DMA,
            pltpu.SemaphoreType.DMA,
            pltpu.SemaphoreType.DMA,
            pltpu.SemaphoreType.DMA,
        ],
    )
    def k(src_h, dst_h, xs_h, o0_h, o1_h, src_v, dst_v, rows0_v, rows1_v,
          acc_s, gsem0, gsem1, ssem0, ssem1):
        c = lax.axis_index("c")
        s = lax.axis_index("s")
        rows = (rows0_v, rows1_v)
        gsems = (gsem0, gsem1)
        ssems = (ssem0, ssem1)

        def zero_body(row, carry):
            for j in range(D // LANES):
                rows0_v[row, pl.ds(j * LANES, LANES)] = (
                    jnp.zeros((LANES,), jnp.float32))
            return carry

        lax.fori_loop(0, CH, zero_body, 0)
        for j in range(rpt // CH):
            pltpu.sync_copy(rows0_v, acc_s.at[pl.ds(s * rpt + j * CH, CH)])
        plsc.subcore_barrier()

        w = c * NS + s
        nh = nch // 2

        def wait_gather(i, b):
            pltpu.make_async_copy(xs_h.at[src_v.at[i]], rows[b],
                                  gsems[b]).wait()

        def wait_scatter(i, b):
            pltpu.make_async_copy(rows[b], acc_s.at[dst_v.at[i]],
                                  ssems[b]).wait()

        for half in range(2):
            hbase = w * nch + half * nh
            pltpu.sync_copy(src_h.at[pl.ds(hbase, nh)], src_v)
            pltpu.sync_copy(dst_h.at[pl.ds(hbase, nh)], dst_v)
            pltpu.async_copy(xs_h.at[src_v.at[0]], rows0_v, gsem0)
            wait_gather(0, 0)
            pltpu.async_copy(rows0_v, acc_s.at[dst_v.at[0]], ssem0, add=True)
            pltpu.async_copy(xs_h.at[src_v.at[1]], rows1_v, gsem1)

            def pair_body(t, carry):
                for i_off, b in ((1, 1), (2, 0)):
                    i = 2 * t + i_off
                    wait_gather(i, b)
                    pltpu.async_copy(rows[b], acc_s.at[dst_v.at[i]],
                                     ssems[b], add=True)
                    wait_scatter(i - 1, 1 - b)
                    pltpu.async_copy(xs_h.at[src_v.at[i + 1]], rows[1 - b],
                                     gsems[1 - b])
                return carry

            lax.fori_loop(0, (nh - 2) // 2, pair_body, 0)
            wait_gather(nh - 1, 1)
            pltpu.async_copy(rows1_v, acc_s.at[dst_v.at[nh - 1]], ssem1,
                             add=True)
            wait_scatter(nh - 2, 0)
            wait_scatter(nh - 1, 1)
        plsc.subcore_barrier()
        for j in range(rpt // CH):
            pltpu.sync_copy(acc_s.at[pl.ds(s * rpt + j * CH, CH)], rows0_v)

            @pl.when(c == 0)
            def _():
                pltpu.sync_copy(rows0_v, o0_h.at[pl.ds(s * rpt + j * CH, CH)])

            @pl.when(c == 1)
            def _():
                pltpu.sync_copy(rows0_v, o1_h.at[pl.ds(s * rpt + j * CH, CH)])

    return k


@functools.cache
def _pre_kernel(N, P, D, br):
    def body(f_ref, do_ref, di_ref, xs_ref, rin_ref, rout_ref):
        r_o = lax.rsqrt(jnp.maximum(do_ref[...], 1.0))
        r_i = lax.rsqrt(jnp.maximum(di_ref[...], 1.0))
        xs_ref[...] = f_ref[...] * r_o
        rin_ref[...] = r_i
        rout_ref[...] = r_o

    rb = pl.BlockSpec((br, D), lambda i: (i, 0))
    cb = pl.BlockSpec((br, 1), lambda i: (i, 0))
    return pl.pallas_call(
        body,
        grid=(N // br,),
        in_specs=[rb, cb, cb],
        out_specs=[rb, cb, cb],
        out_shape=[jax.ShapeDtypeStruct((N, D), jnp.float32),
                   jax.ShapeDtypeStruct((N, 1), jnp.float32),
                   jax.ShapeDtypeStruct((N, 1), jnp.float32)],
    )


@functools.cache
def _mid_kernel(N, P, D, H, br):
    def body(p0_ref, p1_ref, rin_ref, rout_ref, w1_ref, b1_ref, w2_ref,
             out_ref):
        g = (p0_ref[...] + p1_ref[...]) * rin_ref[...]
        y = jnp.where(g > 0, g, jnp.exp(g) - 1.0)
        w1e = w1_ref[0:D, :] + w1_ref[D:2 * D, :]
        h = jnp.dot(y, w1e, preferred_element_type=jnp.float32) + b1_ref[...]
        z = jnp.dot(h, w2_ref[...], preferred_element_type=jnp.float32)
        out_ref[...] = z * rout_ref[...]

    rb = pl.BlockSpec((br, D), lambda i: (i, 0))
    cb = pl.BlockSpec((br, 1), lambda i: (i, 0))
    return pl.pallas_call(
        body,
        grid=(N // br,),
        in_specs=[rb, rb, cb, cb,
                  pl.BlockSpec((2 * D, 2 * H), lambda i: (0, 0)),
                  pl.BlockSpec((1, 2 * H), lambda i: (0, 0)),
                  pl.BlockSpec((2 * H, D), lambda i: (0, 0))],
        out_specs=rb,
        out_shape=jax.ShapeDtypeStruct((N, D), jnp.float32),
    )


@functools.cache
def _fin_kernel(N, P, D, br):
    def body(q0_ref, q1_ref, rin_ref, b2_ref, out_ref):
        out_ref[...] = ((q0_ref[...] + q1_ref[...]) * rin_ref[...]
                        + b2_ref[...])

    rb = pl.BlockSpec((br, D), lambda i: (i, 0))
    cb = pl.BlockSpec((br, 1), lambda i: (i, 0))
    return pl.pallas_call(
        body,
        grid=(N // br,),
        in_specs=[rb, rb, cb, pl.BlockSpec((1, D), lambda i: (0, 0))],
        out_specs=rb,
        out_shape=jax.ShapeDtypeStruct((N, D), jnp.float32),
    )


def kernel(feature, edge_index, W1, b1, W2, b2):
    N, D = feature.shape
    E = edge_index.shape[1]
    H = W2.shape[0] // 2

    rpt = _cdiv(N + 1, NS * CH) * CH
    P = NS * rpt
    E_pad = _cdiv(E, 2 * NC * NS * CH) * (2 * NC * NS * CH)
    br = 400

    npad = E_pad - E
    pad_src = jnp.arange(npad, dtype=jnp.int32) % N
    pad_dst = N + jnp.arange(npad, dtype=jnp.int32) % (P - N)
    src_p = jnp.concatenate([edge_index[0], pad_src]).reshape(E_pad // CH, CH)
    dst_p = jnp.concatenate([edge_index[1], pad_dst]).reshape(E_pad // CH, CH)

    d_out, d_in = _deg_kernel(E, E_pad, P, rpt)(src_p, dst_p)
    xs, r_in, r_out = _pre_kernel(N, P, D, br)(
        feature, d_out[:N].reshape(N, 1), d_in[:N].reshape(N, 1))
    p0, p1 = _prop_kernel(E_pad, N, P, D, rpt)(src_p, dst_p, xs)
    zs = _mid_kernel(N, P, D, H, br)(p0, p1, r_in, r_out, W1,
                                     b1.reshape(1, 2 * H), W2)
    q0, q1 = _prop_kernel(E_pad, N, P, D, rpt)(src_p, dst_p, zs)
    return _fin_kernel(N, P, D, br)(q0, q1, r_in, b2.reshape(1, D))


# speedup vs baseline: 1.1670x; 1.1670x over previous
import functools

import numpy as np
import jax
import jax.numpy as jnp
from jax import lax
from jax.experimental import pallas as pl
from jax.experimental.pallas import tpu as pltpu
from jax.experimental.pallas import tpu_sc as plsc

NC = 2
NS = 16
CH = 128
LANES = 16


def _cdiv(a, b):
    return (a + b - 1) // b


@functools.cache
def _deg_kernel(E, E_pad, P, rpt):
    mesh = plsc.VectorSubcoreMesh(core_axis_name="c", subcore_axis_name="s")
    nreal = E // CH
    nmax = _cdiv(nreal, NS * 8) * 8
    assert nmax * NS <= E_pad // CH

    @functools.partial(
        pl.kernel,
        out_type=(jax.ShapeDtypeStruct((P,), jnp.float32),
                  jax.ShapeDtypeStruct((P,), jnp.float32)),
        mesh=mesh,
        scratch_types=[
            pltpu.VMEM((nmax, CH), jnp.int32),
            pltpu.VMEM((CH,), jnp.float32),
            pltpu.VMEM((rpt,), jnp.float32),
            pltpu.VMEM_SHARED((P,), jnp.float32),
        ],
    )
    def k(src_h, dst_h, dout_h, din_h, idx_v, ones_v, stage_v, acc_s):
        c = lax.axis_index("c")
        s = lax.axis_index("s")
        for j in range(CH // LANES):
            ones_v[pl.ds(j * LANES, LANES)] = jnp.ones((LANES,), jnp.float32)

        def zero_body(i, carry):
            stage_v[pl.ds(i * LANES, LANES)] = jnp.zeros((LANES,), jnp.float32)
            return carry

        lax.fori_loop(0, rpt // LANES, zero_body, 0)
        pltpu.sync_copy(stage_v, acc_s.at[pl.ds(s * rpt, rpt)])
        plsc.subcore_barrier()
        start = s * nmax
        cnt = jnp.clip(nreal - s * nmax, 0, nmax)

        @pl.when(c == 0)
        def _():
            pltpu.sync_copy(src_h.at[pl.ds(start, nmax)], idx_v)

            def body(i, carry):
                pltpu.sync_copy(ones_v, acc_s.at[idx_v.at[i]], add=True)
                return carry

            lax.fori_loop(0, cnt, body, 0)

        @pl.when(c == 1)
        def _():
            pltpu.sync_copy(dst_h.at[pl.ds(start, nmax)], idx_v)

            def body(i, carry):
                pltpu.sync_copy(ones_v, acc_s.at[idx_v.at[i]], add=True)
                return carry

            lax.fori_loop(0, cnt, body, 0)

        plsc.subcore_barrier()
        pltpu.sync_copy(acc_s.at[pl.ds(s * rpt, rpt)], stage_v)

        @pl.when(c == 0)
        def _():
            pltpu.sync_copy(stage_v, dout_h.at[pl.ds(s * rpt, rpt)])

        @pl.when(c == 1)
        def _():
            pltpu.sync_copy(stage_v, din_h.at[pl.ds(s * rpt, rpt)])

    return k


@functools.cache
def _prop_kernel(E_pad, N, P, D, rpt):
    mesh = plsc.VectorSubcoreMesh(core_axis_name="c", subcore_axis_name="s")
    per_worker = E_pad // (NC * NS)
    nch = per_worker // CH
    assert nch % 2 == 0 and nch >= 4

    @functools.partial(
        pl.kernel,
        out_type=(jax.ShapeDtypeStruct((P, D), jnp.float32),
                  jax.ShapeDtypeStruct((P, D), jnp.float32)),
        mesh=mesh,
        scratch_types=[
            pltpu.VMEM((nch // 2, CH), jnp.int32),
            pltpu.VMEM((nch // 2, CH), jnp.int32),
            pltpu.VMEM((CH, D), jnp.float32),
            pltpu.VMEM((CH, D), jnp.float32),
            pltpu.VMEM_SHARED((P, D), jnp.float32),
            pltpu.SemaphoreType.DMA,
            pltpu.SemaphoreType.DMA,
        ],
    )
    def k(src_h, dst_h, xs_h, o0_h, o1_h, src_v, dst_v, rows0_v, rows1_v,
          acc_s, gsem0, gsem1):
        c = lax.axis_index("c")
        s = lax.axis_index("s")
        rows = (rows0_v, rows1_v)
        gsems = (gsem0, gsem1)

        def zero_body(row, carry):
            for j in range(D // LANES):
                rows0_v[row, pl.ds(j * LANES, LANES)] = (
                    jnp.zeros((LANES,), jnp.float32))
            return carry

        lax.fori_loop(0, CH, zero_body, 0)
        for j in range(rpt // CH):
            pltpu.sync_copy(rows0_v, acc_s.at[pl.ds(s * rpt + j * CH, CH)])
        plsc.subcore_barrier()

        w = c * NS + s
        nh = nch // 2
        for half in range(2):
            hbase = w * nch + half * nh
            pltpu.sync_copy(src_h.at[pl.ds(hbase, nh)], src_v)
            pltpu.sync_copy(dst_h.at[pl.ds(hbase, nh)], dst_v)
            pltpu.async_copy(xs_h.at[src_v.at[0]], rows0_v, gsem0)

            def pair_body(t, carry):
                for u in (0, 1):
                    i = 2 * t + u
                    pltpu.async_copy(xs_h.at[src_v.at[i + 1]], rows[1 - u],
                                     gsems[1 - u])
                    pltpu.make_async_copy(xs_h.at[src_v.at[i]], rows[u],
                                          gsems[u]).wait()
                    pltpu.sync_copy(rows[u], acc_s.at[dst_v.at[i]], add=True)
                return carry

            lax.fori_loop(0, (nh - 2) // 2, pair_body, 0)
            pltpu.async_copy(xs_h.at[src_v.at[nh - 1]], rows1_v, gsem1)
            pltpu.make_async_copy(xs_h.at[src_v.at[nh - 2]], rows0_v,
                                  gsem0).wait()
            pltpu.sync_copy(rows0_v, acc_s.at[dst_v.at[nh - 2]], add=True)
            pltpu.make_async_copy(xs_h.at[src_v.at[nh - 1]], rows1_v,
                                  gsem1).wait()
            pltpu.sync_copy(rows1_v, acc_s.at[dst_v.at[nh - 1]], add=True)
        plsc.subcore_barrier()
        for j in range(rpt // CH):
            pltpu.sync_copy(acc_s.at[pl.ds(s * rpt + j * CH, CH)], rows0_v)

            @pl.when(c == 0)
            def _():
                pltpu.sync_copy(rows0_v, o0_h.at[pl.ds(s * rpt + j * CH, CH)])

            @pl.when(c == 1)
            def _():
                pltpu.sync_copy(rows0_v, o1_h.at[pl.ds(s * rpt + j * CH, CH)])

    return k


@functools.cache
def _pre_kernel(P, D, br):
    def body(f_ref, do_ref, di_ref, xs_ref, rin_ref, rout_ref):
        r_o = lax.rsqrt(jnp.maximum(do_ref[...], 1.0))
        r_i = lax.rsqrt(jnp.maximum(di_ref[...], 1.0))
        xs_ref[...] = f_ref[...] * r_o
        rin_ref[...] = r_i
        rout_ref[...] = r_o

    rb = pl.BlockSpec((br, D), lambda i: (i, 0))
    cb = pl.BlockSpec((br, 1), lambda i: (i, 0))
    return pl.pallas_call(
        body,
        grid=(P // br,),
        in_specs=[rb, cb, cb],
        out_specs=[rb, cb, cb],
        out_shape=[jax.ShapeDtypeStruct((P, D), jnp.float32),
                   jax.ShapeDtypeStruct((P, 1), jnp.float32),
                   jax.ShapeDtypeStruct((P, 1), jnp.float32)],
    )


@functools.cache
def _mid_kernel(P, D, H, br):
    def body(p0_ref, p1_ref, rin_ref, rout_ref, w1_ref, b1_ref, w2_ref,
             out_ref):
        g = (p0_ref[...] + p1_ref[...]) * rin_ref[...]
        y = jnp.where(g > 0, g, jnp.exp(g) - 1.0)
        w1e = w1_ref[0:D, :] + w1_ref[D:2 * D, :]
        h = jnp.dot(y, w1e, preferred_element_type=jnp.float32) + b1_ref[...]
        z = jnp.dot(h, w2_ref[...], preferred_element_type=jnp.float32)
        out_ref[...] = z * rout_ref[...]

    rb = pl.BlockSpec((br, D), lambda i: (i, 0))
    cb = pl.BlockSpec((br, 1), lambda i: (i, 0))
    return pl.pallas_call(
        body,
        grid=(P // br,),
        in_specs=[rb, rb, cb, cb,
                  pl.BlockSpec((2 * D, 2 * H), lambda i: (0, 0)),
                  pl.BlockSpec((1, 2 * H), lambda i: (0, 0)),
                  pl.BlockSpec((2 * H, D), lambda i: (0, 0))],
        out_specs=rb,
        out_shape=jax.ShapeDtypeStruct((P, D), jnp.float32),
    )


@functools.cache
def _fin_kernel(P, D, br):
    def body(q0_ref, q1_ref, rin_ref, b2_ref, out_ref):
        out_ref[...] = ((q0_ref[...] + q1_ref[...]) * rin_ref[...]
                        + b2_ref[...])

    rb = pl.BlockSpec((br, D), lambda i: (i, 0))
    cb = pl.BlockSpec((br, 1), lambda i: (i, 0))
    return pl.pallas_call(
        body,
        grid=(P // br,),
        in_specs=[rb, rb, cb, pl.BlockSpec((1, D), lambda i: (0, 0))],
        out_specs=rb,
        out_shape=jax.ShapeDtypeStruct((P, D), jnp.float32),
    )


def kernel(feature, edge_index, W1, b1, W2, b2):
    N, D = feature.shape
    E = edge_index.shape[1]
    H = W2.shape[0] // 2

    rpt = _cdiv(N + 1, NS * CH) * CH
    P = NS * rpt
    E_pad = _cdiv(E, 2 * NC * NS * CH) * (2 * NC * NS * CH)
    br = rpt

    npad = E_pad - E
    pad_idx = jnp.asarray(N + np.arange(npad, dtype=np.int32) % (P - N))
    src_p = jnp.concatenate([edge_index[0], pad_idx]).reshape(E_pad // CH, CH)
    dst_p = jnp.concatenate([edge_index[1], pad_idx]).reshape(E_pad // CH, CH)
    feature_p = jnp.pad(feature, ((0, P - N), (0, 0)))

    d_out, d_in = _deg_kernel(E, E_pad, P, rpt)(src_p, dst_p)
    xs, r_in, r_out = _pre_kernel(P, D, br)(
        feature_p, d_out.reshape(P, 1), d_in.reshape(P, 1))
    p0, p1 = _prop_kernel(E_pad, N, P, D, rpt)(src_p, dst_p, xs)
    zs = _mid_kernel(P, D, H, br)(p0, p1, r_in, r_out, W1,
                                  b1.reshape(1, 2 * H), W2)
    q0, q1 = _prop_kernel(E_pad, N, P, D, rpt)(src_p, dst_p, zs)
    out = _fin_kernel(P, D, br)(q0, q1, r_in, b2.reshape(1, D))
    return out[:N]


# speedup vs baseline: 1.1783x; 1.0098x over previous
import functools

import numpy as np
import jax
import jax.numpy as jnp
from jax import lax
from jax.experimental import pallas as pl
from jax.experimental.pallas import tpu as pltpu
from jax.experimental.pallas import tpu_sc as plsc

NC = 2
NS = 16
CH = 128
LANES = 16


def _cdiv(a, b):
    return (a + b - 1) // b


@functools.cache
def _deg_kernel(E, E_pad, P, rpt):
    mesh = plsc.VectorSubcoreMesh(core_axis_name="c", subcore_axis_name="s")
    nreal = E // CH
    nmax = _cdiv(nreal, NS * 8) * 8
    assert nmax * NS <= E_pad // CH

    @functools.partial(
        pl.kernel,
        out_type=(jax.ShapeDtypeStruct((P,), jnp.float32),
                  jax.ShapeDtypeStruct((P,), jnp.float32)),
        mesh=mesh,
        scratch_types=[
            pltpu.VMEM((nmax, CH), jnp.int32),
            pltpu.VMEM((CH,), jnp.float32),
            pltpu.VMEM((rpt,), jnp.float32),
            pltpu.VMEM_SHARED((P,), jnp.float32),
        ],
    )
    def k(src_h, dst_h, dout_h, din_h, idx_v, ones_v, stage_v, acc_s):
        c = lax.axis_index("c")
        s = lax.axis_index("s")
        for j in range(CH // LANES):
            ones_v[pl.ds(j * LANES, LANES)] = jnp.ones((LANES,), jnp.float32)

        def zero_body(i, carry):
            stage_v[pl.ds(i * LANES, LANES)] = jnp.zeros((LANES,), jnp.float32)
            return carry

        lax.fori_loop(0, rpt // LANES, zero_body, 0)
        pltpu.sync_copy(stage_v, acc_s.at[pl.ds(s * rpt, rpt)])
        plsc.subcore_barrier()
        start = s * nmax
        cnt = jnp.clip(nreal - s * nmax, 0, nmax)

        @pl.when(c == 0)
        def _():
            pltpu.sync_copy(src_h.at[pl.ds(start, nmax)], idx_v)

            def body(i, carry):
                pltpu.sync_copy(ones_v, acc_s.at[idx_v.at[i]], add=True)
                return carry

            lax.fori_loop(0, cnt, body, 0)

        @pl.when(c == 1)
        def _():
            pltpu.sync_copy(dst_h.at[pl.ds(start, nmax)], idx_v)

            def body(i, carry):
                pltpu.sync_copy(ones_v, acc_s.at[idx_v.at[i]], add=True)
                return carry

            lax.fori_loop(0, cnt, body, 0)

        plsc.subcore_barrier()
        pltpu.sync_copy(acc_s.at[pl.ds(s * rpt, rpt)], stage_v)

        @pl.when(c == 0)
        def _():
            pltpu.sync_copy(stage_v, dout_h.at[pl.ds(s * rpt, rpt)])

        @pl.when(c == 1)
        def _():
            pltpu.sync_copy(stage_v, din_h.at[pl.ds(s * rpt, rpt)])

    return k


@functools.cache
def _prop_kernel(E_pad, N, P, D, rpt):
    mesh = plsc.VectorSubcoreMesh(core_axis_name="c", subcore_axis_name="s")
    per_worker = E_pad // (NC * NS)
    nch = per_worker // CH
    assert nch % 2 == 0 and nch >= 4

    @functools.partial(
        pl.kernel,
        out_type=(jax.ShapeDtypeStruct((P, D), jnp.float32),
                  jax.ShapeDtypeStruct((P, D), jnp.float32)),
        mesh=mesh,
        scratch_types=[
            pltpu.VMEM((nch // 2, CH), jnp.int32),
            pltpu.VMEM((nch // 2, CH), jnp.int32),
            pltpu.VMEM((CH, D), jnp.float32),
            pltpu.VMEM((CH, D), jnp.float32),
            pltpu.VMEM_SHARED((P, D), jnp.float32),
            pltpu.SemaphoreType.DMA,
            pltpu.SemaphoreType.DMA,
        ],
    )
    def k(src_h, dst_h, xs_h, o0_h, o1_h, src_v, dst_v, rows0_v, rows1_v,
          acc_s, gsem0, gsem1):
        c = lax.axis_index("c")
        s = lax.axis_index("s")
        rows = (rows0_v, rows1_v)
        gsems = (gsem0, gsem1)

        def zero_body(row, carry):
            for j in range(D // LANES):
                rows0_v[row, pl.ds(j * LANES, LANES)] = (
                    jnp.zeros((LANES,), jnp.float32))
            return carry

        lax.fori_loop(0, CH, zero_body, 0)
        for j in range(rpt // CH):
            pltpu.sync_copy(rows0_v, acc_s.at[pl.ds(s * rpt + j * CH, CH)])
        plsc.subcore_barrier()

        w = c * NS + s
        nh = nch // 2
        for half in range(2):
            hbase = w * nch + half * nh
            pltpu.sync_copy(src_h.at[pl.ds(hbase, nh)], src_v)
            pltpu.sync_copy(dst_h.at[pl.ds(hbase, nh)], dst_v)
            pltpu.async_copy(xs_h.at[src_v.at[0]], rows0_v, gsem0)

            def pair_body(t, carry):
                for u in (0, 1):
                    i = 2 * t + u
                    pltpu.async_copy(xs_h.at[src_v.at[i + 1]], rows[1 - u],
                                     gsems[1 - u])
                    pltpu.make_async_copy(xs_h.at[src_v.at[i]], rows[u],
                                          gsems[u]).wait()
                    pltpu.sync_copy(rows[u], acc_s.at[dst_v.at[i]], add=True)
                return carry

            lax.fori_loop(0, (nh - 2) // 2, pair_body, 0)
            pltpu.async_copy(xs_h.at[src_v.at[nh - 1]], rows1_v, gsem1)
            pltpu.make_async_copy(xs_h.at[src_v.at[nh - 2]], rows0_v,
                                  gsem0).wait()
            pltpu.sync_copy(rows0_v, acc_s.at[dst_v.at[nh - 2]], add=True)
            pltpu.make_async_copy(xs_h.at[src_v.at[nh - 1]], rows1_v,
                                  gsem1).wait()
            pltpu.sync_copy(rows1_v, acc_s.at[dst_v.at[nh - 1]], add=True)
        plsc.subcore_barrier()
        ndr = rpt // CH
        for j in range(ndr):
            b = rows[j % 2]
            sl = pl.ds(s * rpt + j * CH, CH)
            if j >= 2:
                psl = pl.ds(s * rpt + (j - 2) * CH, CH)

                @pl.when(c == 0)
                def _():
                    pltpu.make_async_copy(rows[j % 2], o0_h.at[psl],
                                          gsems[j % 2]).wait()

                @pl.when(c == 1)
                def _():
                    pltpu.make_async_copy(rows[j % 2], o1_h.at[psl],
                                          gsems[j % 2]).wait()

            pltpu.sync_copy(acc_s.at[sl], b)

            @pl.when(c == 0)
            def _():
                pltpu.async_copy(b, o0_h.at[sl], gsems[j % 2])

            @pl.when(c == 1)
            def _():
                pltpu.async_copy(b, o1_h.at[sl], gsems[j % 2])

        for j in range(max(ndr - 2, 0), ndr):
            sl = pl.ds(s * rpt + j * CH, CH)

            @pl.when(c == 0)
            def _():
                pltpu.make_async_copy(rows[j % 2], o0_h.at[sl],
                                      gsems[j % 2]).wait()

            @pl.when(c == 1)
            def _():
                pltpu.make_async_copy(rows[j % 2], o1_h.at[sl],
                                      gsems[j % 2]).wait()

    return k


@functools.cache
def _pre_kernel(P, D, br):
    def body(f_ref, do_ref, di_ref, xs_ref, rin_ref, rout_ref):
        r_o = lax.rsqrt(jnp.maximum(do_ref[...], 1.0))
        r_i = lax.rsqrt(jnp.maximum(di_ref[...], 1.0))
        xs_ref[...] = f_ref[...] * r_o
        rin_ref[...] = r_i
        rout_ref[...] = r_o

    rb = pl.BlockSpec((br, D), lambda i: (i, 0))
    cb = pl.BlockSpec((br, 1), lambda i: (i, 0))
    return pl.pallas_call(
        body,
        grid=(P // br,),
        in_specs=[rb, cb, cb],
        out_specs=[rb, cb, cb],
        out_shape=[jax.ShapeDtypeStruct((P, D), jnp.float32),
                   jax.ShapeDtypeStruct((P, 1), jnp.float32),
                   jax.ShapeDtypeStruct((P, 1), jnp.float32)],
    )


@functools.cache
def _mid_kernel(P, D, H, br):
    def body(p0_ref, p1_ref, rin_ref, rout_ref, w1_ref, b1_ref, w2_ref,
             out_ref):
        g = (p0_ref[...] + p1_ref[...]) * rin_ref[...]
        y = jnp.where(g > 0, g, jnp.exp(g) - 1.0)
        w1e = w1_ref[0:D, :] + w1_ref[D:2 * D, :]
        h = jnp.dot(y, w1e, preferred_element_type=jnp.float32) + b1_ref[...]
        z = jnp.dot(h, w2_ref[...], preferred_element_type=jnp.float32)
        out_ref[...] = z * rout_ref[...]

    rb = pl.BlockSpec((br, D), lambda i: (i, 0))
    cb = pl.BlockSpec((br, 1), lambda i: (i, 0))
    return pl.pallas_call(
        body,
        grid=(P // br,),
        in_specs=[rb, rb, cb, cb,
                  pl.BlockSpec((2 * D, 2 * H), lambda i: (0, 0)),
                  pl.BlockSpec((1, 2 * H), lambda i: (0, 0)),
                  pl.BlockSpec((2 * H, D), lambda i: (0, 0))],
        out_specs=rb,
        out_shape=jax.ShapeDtypeStruct((P, D), jnp.float32),
    )


@functools.cache
def _fin_kernel(P, D, br):
    def body(q0_ref, q1_ref, rin_ref, b2_ref, out_ref):
        out_ref[...] = ((q0_ref[...] + q1_ref[...]) * rin_ref[...]
                        + b2_ref[...])

    rb = pl.BlockSpec((br, D), lambda i: (i, 0))
    cb = pl.BlockSpec((br, 1), lambda i: (i, 0))
    return pl.pallas_call(
        body,
        grid=(P // br,),
        in_specs=[rb, rb, cb, pl.BlockSpec((1, D), lambda i: (0, 0))],
        out_specs=rb,
        out_shape=jax.ShapeDtypeStruct((P, D), jnp.float32),
    )


def kernel(feature, edge_index, W1, b1, W2, b2):
    N, D = feature.shape
    E = edge_index.shape[1]
    H = W2.shape[0] // 2

    rpt = _cdiv(N + 1, NS * CH) * CH
    P = NS * rpt
    E_pad = _cdiv(E, 2 * NC * NS * CH) * (2 * NC * NS * CH)
    br = rpt

    npad = E_pad - E
    pad_idx = jnp.asarray(N + np.arange(npad, dtype=np.int32) % (P - N))
    src_p = jnp.concatenate([edge_index[0], pad_idx]).reshape(E_pad // CH, CH)
    dst_p = jnp.concatenate([edge_index[1], pad_idx]).reshape(E_pad // CH, CH)
    feature_p = jnp.pad(feature, ((0, P - N), (0, 0)))

    d_out, d_in = _deg_kernel(E, E_pad, P, rpt)(src_p, dst_p)
    xs, r_in, r_out = _pre_kernel(P, D, br)(
        feature_p, d_out.reshape(P, 1), d_in.reshape(P, 1))
    p0, p1 = _prop_kernel(E_pad, N, P, D, rpt)(src_p, dst_p, xs)
    zs = _mid_kernel(P, D, H, br)(p0, p1, r_in, r_out, W1,
                                  b1.reshape(1, 2 * H), W2)
    q0, q1 = _prop_kernel(E_pad, N, P, D, rpt)(src_p, dst_p, zs)
    out = _fin_kernel(P, D, br)(q0, q1, r_in, b2.reshape(1, D))
    return out[:N]


# speedup vs baseline: 1.1796x; 1.0010x over previous
import functools

import numpy as np
import jax
import jax.numpy as jnp
from jax import lax
from jax.experimental import pallas as pl
from jax.experimental.pallas import tpu as pltpu
from jax.experimental.pallas import tpu_sc as plsc

NC = 2
NS = 16
CH = 128
LANES = 16


def _cdiv(a, b):
    return (a + b - 1) // b


@functools.cache
def _deg_kernel(E, E_pad, P, rpt):
    mesh = plsc.VectorSubcoreMesh(core_axis_name="c", subcore_axis_name="s")
    nreal = E // CH
    nmax = _cdiv(nreal, NS * 8) * 8
    assert nmax * NS <= E_pad // CH

    @functools.partial(
        pl.kernel,
        out_type=(jax.ShapeDtypeStruct((P,), jnp.float32),
                  jax.ShapeDtypeStruct((P,), jnp.float32)),
        mesh=mesh,
        scratch_types=[
            pltpu.VMEM((nmax, CH), jnp.int32),
            pltpu.VMEM((CH,), jnp.float32),
            pltpu.VMEM((rpt,), jnp.float32),
            pltpu.VMEM_SHARED((P,), jnp.float32),
        ],
    )
    def k(src_h, dst_h, dout_h, din_h, idx_v, ones_v, stage_v, acc_s):
        c = lax.axis_index("c")
        s = lax.axis_index("s")
        for j in range(CH // LANES):
            ones_v[pl.ds(j * LANES, LANES)] = jnp.ones((LANES,), jnp.float32)

        def zero_body(i, carry):
            stage_v[pl.ds(i * LANES, LANES)] = jnp.zeros((LANES,), jnp.float32)
            return carry

        lax.fori_loop(0, rpt // LANES, zero_body, 0)
        pltpu.sync_copy(stage_v, acc_s.at[pl.ds(s * rpt, rpt)])
        plsc.subcore_barrier()
        start = s * nmax
        cnt = jnp.clip(nreal - s * nmax, 0, nmax)

        @pl.when(c == 0)
        def _():
            pltpu.sync_copy(src_h.at[pl.ds(start, nmax)], idx_v)

            def body(i, carry):
                pltpu.sync_copy(ones_v, acc_s.at[idx_v.at[i]], add=True)
                return carry

            lax.fori_loop(0, cnt, body, 0)

        @pl.when(c == 1)
        def _():
            pltpu.sync_copy(dst_h.at[pl.ds(start, nmax)], idx_v)

            def body(i, carry):
                pltpu.sync_copy(ones_v, acc_s.at[idx_v.at[i]], add=True)
                return carry

            lax.fori_loop(0, cnt, body, 0)

        plsc.subcore_barrier()
        pltpu.sync_copy(acc_s.at[pl.ds(s * rpt, rpt)], stage_v)

        @pl.when(c == 0)
        def _():
            pltpu.sync_copy(stage_v, dout_h.at[pl.ds(s * rpt, rpt)])

        @pl.when(c == 1)
        def _():
            pltpu.sync_copy(stage_v, din_h.at[pl.ds(s * rpt, rpt)])

    return k


@functools.cache
def _prop_kernel(E_pad, N, P, D, rpt):
    mesh = plsc.VectorSubcoreMesh(core_axis_name="c", subcore_axis_name="s")
    per_worker = E_pad // (NC * NS)
    nch = per_worker // CH
    assert nch % 2 == 0 and nch >= 4

    @functools.partial(
        pl.kernel,
        out_type=(jax.ShapeDtypeStruct((P, D), jnp.float32),
                  jax.ShapeDtypeStruct((P, D), jnp.float32)),
        mesh=mesh,
        scratch_types=[
            pltpu.VMEM((nch // 2, CH), jnp.int32),
            pltpu.VMEM((nch // 2, CH), jnp.int32),
            pltpu.VMEM((CH, D), jnp.float32),
            pltpu.VMEM((CH, D), jnp.float32),
            pltpu.VMEM_SHARED((P, D), jnp.float32),
            pltpu.SemaphoreType.DMA,
            pltpu.SemaphoreType.DMA,
        ],
    )
    def k(src_h, dst_h, xs_h, o0_h, o1_h, src_v, dst_v, rows0_v, rows1_v,
          acc_s, gsem0, gsem1):
        c = lax.axis_index("c")
        s = lax.axis_index("s")
        rows = (rows0_v, rows1_v)
        gsems = (gsem0, gsem1)

        def zero_body(row, carry):
            for j in range(D // LANES):
                rows0_v[row, pl.ds(j * LANES, LANES)] = (
                    jnp.zeros((LANES,), jnp.float32))
            return carry

        lax.fori_loop(0, CH, zero_body, 0)
        for j in range(rpt // CH):
            pltpu.async_copy(rows0_v, acc_s.at[pl.ds(s * rpt + j * CH, CH)],
                             gsem0)
        for j in range(rpt // CH):
            pltpu.make_async_copy(
                rows0_v, acc_s.at[pl.ds(s * rpt + j * CH, CH)], gsem0).wait()
        plsc.subcore_barrier()

        w = c * NS + s
        nh = nch // 2
        for half in range(2):
            hbase = w * nch + half * nh
            pltpu.sync_copy(src_h.at[pl.ds(hbase, nh)], src_v)
            pltpu.sync_copy(dst_h.at[pl.ds(hbase, nh)], dst_v)
            pltpu.async_copy(xs_h.at[src_v.at[0]], rows0_v, gsem0)

            def pair_body(t, carry):
                for u in (0, 1):
                    i = 2 * t + u
                    pltpu.async_copy(xs_h.at[src_v.at[i + 1]], rows[1 - u],
                                     gsems[1 - u])
                    pltpu.make_async_copy(xs_h.at[src_v.at[i]], rows[u],
                                          gsems[u]).wait()
                    pltpu.sync_copy(rows[u], acc_s.at[dst_v.at[i]], add=True)
                return carry

            lax.fori_loop(0, (nh - 2) // 2, pair_body, 0)
            pltpu.async_copy(xs_h.at[src_v.at[nh - 1]], rows1_v, gsem1)
            pltpu.make_async_copy(xs_h.at[src_v.at[nh - 2]], rows0_v,
                                  gsem0).wait()
            pltpu.sync_copy(rows0_v, acc_s.at[dst_v.at[nh - 2]], add=True)
            pltpu.make_async_copy(xs_h.at[src_v.at[nh - 1]], rows1_v,
                                  gsem1).wait()
            pltpu.sync_copy(rows1_v, acc_s.at[dst_v.at[nh - 1]], add=True)
        plsc.subcore_barrier()
        ndr = rpt // CH
        for j in range(ndr):
            b = rows[j % 2]
            sl = pl.ds(s * rpt + j * CH, CH)
            if j >= 2:
                psl = pl.ds(s * rpt + (j - 2) * CH, CH)

                @pl.when(c == 0)
                def _():
                    pltpu.make_async_copy(rows[j % 2], o0_h.at[psl],
                                          gsems[j % 2]).wait()

                @pl.when(c == 1)
                def _():
                    pltpu.make_async_copy(rows[j % 2], o1_h.at[psl],
                                          gsems[j % 2]).wait()

            pltpu.sync_copy(acc_s.at[sl], b)

            @pl.when(c == 0)
            def _():
                pltpu.async_copy(b, o0_h.at[sl], gsems[j % 2])

            @pl.when(c == 1)
            def _():
                pltpu.async_copy(b, o1_h.at[sl], gsems[j % 2])

        for j in range(max(ndr - 2, 0), ndr):
            sl = pl.ds(s * rpt + j * CH, CH)

            @pl.when(c == 0)
            def _():
                pltpu.make_async_copy(rows[j % 2], o0_h.at[sl],
                                      gsems[j % 2]).wait()

            @pl.when(c == 1)
            def _():
                pltpu.make_async_copy(rows[j % 2], o1_h.at[sl],
                                      gsems[j % 2]).wait()

    return k


@functools.cache
def _pre_kernel(P, D, br):
    def body(f_ref, do_ref, di_ref, xs_ref, rin_ref, rout_ref):
        r_o = lax.rsqrt(jnp.maximum(do_ref[...], 1.0))
        r_i = lax.rsqrt(jnp.maximum(di_ref[...], 1.0))
        xs_ref[...] = f_ref[...] * r_o
        rin_ref[...] = r_i
        rout_ref[...] = r_o

    rb = pl.BlockSpec((br, D), lambda i: (i, 0))
    cb = pl.BlockSpec((br, 1), lambda i: (i, 0))
    return pl.pallas_call(
        body,
        grid=(P // br,),
        in_specs=[rb, cb, cb],
        out_specs=[rb, cb, cb],
        out_shape=[jax.ShapeDtypeStruct((P, D), jnp.float32),
                   jax.ShapeDtypeStruct((P, 1), jnp.float32),
                   jax.ShapeDtypeStruct((P, 1), jnp.float32)],
    )


@functools.cache
def _mid_kernel(P, D, H, br):
    def body(p0_ref, p1_ref, rin_ref, rout_ref, w1_ref, b1_ref, w2_ref,
             out_ref):
        g = (p0_ref[...] + p1_ref[...]) * rin_ref[...]
        y = jnp.where(g > 0, g, jnp.exp(g) - 1.0)
        w1e = w1_ref[0:D, :] + w1_ref[D:2 * D, :]
        h = jnp.dot(y, w1e, preferred_element_type=jnp.float32) + b1_ref[...]
        z = jnp.dot(h, w2_ref[...], preferred_element_type=jnp.float32)
        out_ref[...] = z * rout_ref[...]

    rb = pl.BlockSpec((br, D), lambda i: (i, 0))
    cb = pl.BlockSpec((br, 1), lambda i: (i, 0))
    return pl.pallas_call(
        body,
        grid=(P // br,),
        in_specs=[rb, rb, cb, cb,
                  pl.BlockSpec((2 * D, 2 * H), lambda i: (0, 0)),
                  pl.BlockSpec((1, 2 * H), lambda i: (0, 0)),
                  pl.BlockSpec((2 * H, D), lambda i: (0, 0))],
        out_specs=rb,
        out_shape=jax.ShapeDtypeStruct((P, D), jnp.float32),
    )


@functools.cache
def _fin_kernel(P, D, br):
    def body(q0_ref, q1_ref, rin_ref, b2_ref, out_ref):
        out_ref[...] = ((q0_ref[...] + q1_ref[...]) * rin_ref[...]
                        + b2_ref[...])

    rb = pl.BlockSpec((br, D), lambda i: (i, 0))
    cb = pl.BlockSpec((br, 1), lambda i: (i, 0))
    return pl.pallas_call(
        body,
        grid=(P // br,),
        in_specs=[rb, rb, cb, pl.BlockSpec((1, D), lambda i: (0, 0))],
        out_specs=rb,
        out_shape=jax.ShapeDtypeStruct((P, D), jnp.float32),
    )


def kernel(feature, edge_index, W1, b1, W2, b2):
    N, D = feature.shape
    E = edge_index.shape[1]
    H = W2.shape[0] // 2

    rpt = _cdiv(N + 1, NS * CH) * CH
    P = NS * rpt
    E_pad = _cdiv(E, 2 * NC * NS * CH) * (2 * NC * NS * CH)
    br = rpt

    npad = E_pad - E
    pad_idx = jnp.asarray(N + np.arange(npad, dtype=np.int32) % (P - N))
    src_p = jnp.concatenate([edge_index[0], pad_idx]).reshape(E_pad // CH, CH)
    dst_p = jnp.concatenate([edge_index[1], pad_idx]).reshape(E_pad // CH, CH)
    feature_p = jnp.pad(feature, ((0, P - N), (0, 0)))

    d_out, d_in = _deg_kernel(E, E_pad, P, rpt)(src_p, dst_p)
    xs, r_in, r_out = _pre_kernel(P, D, br)(
        feature_p, d_out.reshape(P, 1), d_in.reshape(P, 1))
    p0, p1 = _prop_kernel(E_pad, N, P, D, rpt)(src_p, dst_p, xs)
    zs = _mid_kernel(P, D, H, br)(p0, p1, r_in, r_out, W1,
                                  b1.reshape(1, 2 * H), W2)
    q0, q1 = _prop_kernel(E_pad, N, P, D, rpt)(src_p, dst_p, zs)
    out = _fin_kernel(P, D, br)(q0, q1, r_in, b2.reshape(1, D))
    return out[:N]
